# baseline (speedup 1.0000x reference)
import functools

import jax
import jax.numpy as jnp
from jax import lax
from jax.experimental import pallas as pl
from jax.experimental.pallas import tpu as pltpu
from jax.experimental.pallas import tpu_sc as plsc

B, N, D, KNN, GR = 4, 4096, 64, 16, 32
HD = 3 * GR
TILE = 256
BN = B * N
NC, NS = 2, 16
NW = NC * NS
PPW = BN // NW
CP = 8
NCHUNK = PPW // CP


def _hbase_body(x_ref, wf_ref, bf_ref, wm_ref, bm_ref, wl_ref, bl_ref,
                h_ref, base_ref):
    xb = x_ref[0]
    wf = wf_ref[...]
    wa, wb, wc = wf[0:D, :], wf[D:2 * D, :], wf[2 * D:3 * D, :]
    wm1, wm2 = wm_ref[0:GR, :], wm_ref[GR:GR + D, :]
    wl1, wl2, wl3 = wl_ref[0:GR, :], wl_ref[GR:2 * GR, :], wl_ref[2 * GR:, :]
    f32 = jnp.float32
    dot = functools.partial(jnp.dot, preferred_element_type=f32,
                            precision=lax.Precision.HIGHEST)
    u1 = wb + wc
    u2 = dot(u1, wm1)
    u3 = dot(u2, wl1) + dot(u1, wl2)
    a = dot(xb, wa - wc) + bf_ref[...]
    b = dot(a, wm1) + dot(xb, wm2) + bm_ref[...]
    c = dot(b, wl1) + dot(a, wl2) + dot(xb, wl3) + bl_ref[...]
    h_ref[0] = jnp.concatenate(
        [dot(xb, u3), dot(xb, u2), dot(xb, u1),
         jnp.zeros((N, 128 - HD), f32)], axis=1)
    base_ref[0] = jnp.concatenate([c, b, a], axis=1)


def _knn_body(post_ref, idx_ref):
    bb = pl.program_id(0)
    i = pl.program_id(1)
    rows = pl.ds(i * TILE, TILE)
    pos_t = post_ref[0]
    f32 = jnp.float32

    d2_all = jnp.sum(pos_t * pos_t, axis=0)[None, :]
    pos_rows = post_ref[0, :, rows]
    cross = lax.dot_general(pos_rows.astype(jnp.bfloat16),
                            pos_t.astype(jnp.bfloat16),
                            (((0,), (0,)), ((), ())),
                            preferred_element_type=f32)
    d2_rows = jnp.sum(pos_rows * pos_rows, axis=0)[:, None]
    dist = d2_rows + d2_all - 2.0 * cross

    iota_col = lax.broadcasted_iota(jnp.int32, (TILE, N), 1)
    inf = jnp.float32(jnp.inf)

    def pick(dcur):
        rowmin = jnp.min(dcur, axis=1, keepdims=True)
        cand = jnp.where(dcur == rowmin, iota_col, N)
        idxm = jnp.min(cand, axis=1, keepdims=True)
        return idxm, iota_col == idxm

    idxm, onehot = pick(dist)
    dist = jnp.where(onehot, inf, dist)

    cols = []
    for _ in range(KNN):
        idxm, onehot = pick(dist)
        cols.append(idxm + bb * N)
        dist = jnp.where(onehot, inf, dist)
    idx_ref[0] = jnp.concatenate(cols, axis=1)


def _sc_gather_body(h_hbm, idx_hbm, base_hbm, out_hbm, idx_v, rows_v, acc_v,
                    sem):
    wid = lax.axis_index("s") * NC + lax.axis_index("c")

    def chunk(ci, carry):
        point_base = wid * PPW + ci * CP
        pltpu.sync_copy(idx_hbm.at[pl.ds(point_base * KNN, CP * KNN)], idx_v)
        pltpu.async_copy(h_hbm.at[idx_v], rows_v, sem).wait()
        pltpu.sync_copy(base_hbm.at[pl.ds(point_base, CP)], acc_v)
        for p in range(CP):
            for c in range(HD // 16):
                sl = pl.ds(c * 16, 16)
                m = rows_v[p * KNN, sl]
                for n in range(1, KNN):
                    m = jnp.maximum(m, rows_v[p * KNN + n, sl])
                acc_v[p, sl] = acc_v[p, sl] + m
        pltpu.sync_copy(acc_v, out_hbm.at[pl.ds(point_base, CP)])
        return carry

    lax.fori_loop(0, NCHUNK, chunk, 0)


def kernel(x, pos, W_first, b_first, W_mid, b_mid, W_last, b_last):
    f32 = jnp.float32
    b_first2 = b_first.reshape(1, GR)
    b_mid2 = b_mid.reshape(1, GR)
    b_last2 = b_last.reshape(1, GR)

    h, base = pl.pallas_call(
        _hbase_body,
        grid=(B,),
        in_specs=[
            pl.BlockSpec((1, N, D), lambda b: (b, 0, 0)),
            pl.BlockSpec((3 * D, GR), lambda b: (0, 0)),
            pl.BlockSpec((1, GR), lambda b: (0, 0)),
            pl.BlockSpec((D + GR, GR), lambda b: (0, 0)),
            pl.BlockSpec((1, GR), lambda b: (0, 0)),
            pl.BlockSpec((D + 2 * GR, GR), lambda b: (0, 0)),
            pl.BlockSpec((1, GR), lambda b: (0, 0)),
        ],
        out_specs=[
            pl.BlockSpec((1, N, 128), lambda b: (b, 0, 0)),
            pl.BlockSpec((1, N, HD), lambda b: (b, 0, 0)),
        ],
        out_shape=[
            jax.ShapeDtypeStruct((B, N, 128), f32),
            jax.ShapeDtypeStruct((B, N, HD), f32),
        ],
    )(x, W_first, b_first2, W_mid, b_mid2, W_last, b_last2)

    pos_t = jnp.concatenate(
        [pos.transpose(0, 2, 1), jnp.zeros((B, 5, N), f32)], axis=1)

    idx = pl.pallas_call(
        _knn_body,
        grid=(B, N // TILE),
        in_specs=[pl.BlockSpec((1, 8, N), lambda b, i: (b, 0, 0))],
        out_specs=pl.BlockSpec((1, TILE, KNN), lambda b, i: (b, i, 0)),
        out_shape=jax.ShapeDtypeStruct((B, N, KNN), jnp.int32),
        compiler_params=pltpu.CompilerParams(
            dimension_semantics=("parallel", "arbitrary")),
    )(pos_t)

    sc_gather = functools.partial(
        pl.kernel,
        out_type=jax.ShapeDtypeStruct((BN, HD), f32),
        mesh=plsc.VectorSubcoreMesh(core_axis_name="c", subcore_axis_name="s"),
        scratch_types=[
            pltpu.VMEM((CP * KNN,), jnp.int32),
            pltpu.VMEM((CP * KNN, 128), f32),
            pltpu.VMEM((CP, HD), f32),
            pltpu.SemaphoreType.DMA,
        ],
    )(_sc_gather_body)

    out96 = sc_gather(h.reshape(BN, 128), idx.reshape(BN * KNN),
                      base.reshape(BN, HD))
    return jnp.concatenate([out96.reshape(B, N, HD), x], axis=-1)


# speedup vs baseline: 1.2761x; 1.2761x over previous
import functools

import jax
import jax.numpy as jnp
from jax import lax
from jax.experimental import pallas as pl
from jax.experimental.pallas import tpu as pltpu
from jax.experimental.pallas import tpu_sc as plsc

B, N, D, KNN, GR = 4, 4096, 64, 16, 32
HD = 3 * GR
TILE = 256
NC, NS = 2, 16
NW = NC * NS
PPW = N // NW
CP = 8
NCHUNK = PPW // CP


def _hbase_body(x_ref, wf_ref, bf_ref, wm_ref, bm_ref, wl_ref, bl_ref,
                h_ref, base_ref):
    xb = x_ref[0]
    wf = wf_ref[...]
    wa, wb, wc = wf[0:D, :], wf[D:2 * D, :], wf[2 * D:3 * D, :]
    wm1, wm2 = wm_ref[0:GR, :], wm_ref[GR:GR + D, :]
    wl1, wl2, wl3 = wl_ref[0:GR, :], wl_ref[GR:2 * GR, :], wl_ref[2 * GR:, :]
    f32 = jnp.float32
    dot = functools.partial(jnp.dot, preferred_element_type=f32,
                            precision=lax.Precision.HIGHEST)
    u1 = wb + wc
    u2 = dot(u1, wm1)
    u3 = dot(u2, wl1) + dot(u1, wl2)
    a = dot(xb, wa - wc) + bf_ref[...]
    b = dot(a, wm1) + dot(xb, wm2) + bm_ref[...]
    c = dot(b, wl1) + dot(a, wl2) + dot(xb, wl3) + bl_ref[...]
    h_ref[0] = jnp.concatenate(
        [dot(xb, u3), dot(xb, u2), dot(xb, u1),
         jnp.zeros((N, 128 - HD), f32)], axis=1)
    base_ref[0] = jnp.concatenate([c, b, a], axis=1)


def _knn_body(post_ref, idx_ref):
    i = pl.program_id(0)
    rows = pl.ds(i * TILE, TILE)
    pos_t = post_ref[...]
    f32 = jnp.float32

    d2_all = jnp.sum(pos_t * pos_t, axis=0)[None, :]
    pos_rows = post_ref[:, rows]
    cross = lax.dot_general(pos_rows.astype(jnp.bfloat16),
                            pos_t.astype(jnp.bfloat16),
                            (((0,), (0,)), ((), ())),
                            preferred_element_type=f32)
    d2_rows = jnp.sum(pos_rows * pos_rows, axis=0)[:, None]
    dist = d2_rows + d2_all - 2.0 * cross

    iota_col = lax.broadcasted_iota(jnp.int32, (TILE, N), 1)
    inf = jnp.float32(jnp.inf)

    rowmin = jnp.min(dist, axis=1, keepdims=True)
    dist = jnp.where(dist == rowmin, inf, dist)

    cols = []
    for _ in range(KNN):
        rowmin = jnp.min(dist, axis=1, keepdims=True)
        eq = dist == rowmin
        cand = jnp.where(eq, iota_col, N)
        cols.append(jnp.min(cand, axis=1, keepdims=True))
        dist = jnp.where(eq, inf, dist)
    idx_ref[...] = jnp.concatenate(cols, axis=1)


def _sc_gather_body(h_hbm, idx_hbm, base_hbm, out_hbm, idx_v, rows_v, acc_v,
                    sem):
    wid = lax.axis_index("s") * NC + lax.axis_index("c")

    def chunk(ci, carry):
        point_base = wid * PPW + ci * CP
        pltpu.sync_copy(idx_hbm.at[pl.ds(point_base * KNN, CP * KNN)], idx_v)
        pltpu.async_copy(h_hbm.at[idx_v], rows_v, sem).wait()
        pltpu.sync_copy(base_hbm.at[pl.ds(point_base, CP)], acc_v)
        for p in range(CP):
            for c in range(HD // 16):
                sl = pl.ds(c * 16, 16)
                m = rows_v[p * KNN, sl]
                for n in range(1, KNN):
                    m = jnp.maximum(m, rows_v[p * KNN + n, sl])
                acc_v[p, sl] = acc_v[p, sl] + m
        pltpu.sync_copy(acc_v, out_hbm.at[pl.ds(point_base, CP)])
        return carry

    lax.fori_loop(0, NCHUNK, chunk, 0)


def kernel(x, pos, W_first, b_first, W_mid, b_mid, W_last, b_last):
    f32 = jnp.float32
    b_first2 = b_first.reshape(1, GR)
    b_mid2 = b_mid.reshape(1, GR)
    b_last2 = b_last.reshape(1, GR)

    h, base = pl.pallas_call(
        _hbase_body,
        grid=(B,),
        in_specs=[
            pl.BlockSpec((1, N, D), lambda b: (b, 0, 0)),
            pl.BlockSpec((3 * D, GR), lambda b: (0, 0)),
            pl.BlockSpec((1, GR), lambda b: (0, 0)),
            pl.BlockSpec((D + GR, GR), lambda b: (0, 0)),
            pl.BlockSpec((1, GR), lambda b: (0, 0)),
            pl.BlockSpec((D + 2 * GR, GR), lambda b: (0, 0)),
            pl.BlockSpec((1, GR), lambda b: (0, 0)),
        ],
        out_specs=[
            pl.BlockSpec((1, N, 128), lambda b: (b, 0, 0)),
            pl.BlockSpec((1, N, HD), lambda b: (b, 0, 0)),
        ],
        out_shape=[
            jax.ShapeDtypeStruct((B, N, 128), f32),
            jax.ShapeDtypeStruct((B, N, HD), f32),
        ],
    )(x, W_first, b_first2, W_mid, b_mid2, W_last, b_last2)

    pos_t = jnp.concatenate(
        [pos.transpose(0, 2, 1), jnp.zeros((B, 5, N), f32)], axis=1)

    knn_call = pl.pallas_call(
        _knn_body,
        grid=(N // TILE,),
        in_specs=[pl.BlockSpec((8, N), lambda i: (0, 0))],
        out_specs=pl.BlockSpec((TILE, KNN), lambda i: (i, 0)),
        out_shape=jax.ShapeDtypeStruct((N, KNN), jnp.int32),
        compiler_params=pltpu.CompilerParams(
            dimension_semantics=("arbitrary",)),
    )

    sc_gather = functools.partial(
        pl.kernel,
        out_type=jax.ShapeDtypeStruct((N, HD), f32),
        mesh=plsc.VectorSubcoreMesh(core_axis_name="c", subcore_axis_name="s"),
        scratch_types=[
            pltpu.VMEM((CP * KNN,), jnp.int32),
            pltpu.VMEM((CP * KNN, 128), f32),
            pltpu.VMEM((CP, HD), f32),
            pltpu.SemaphoreType.DMA,
        ],
    )(_sc_gather_body)

    outs = []
    for bb in range(B):
        idx_b = knn_call(pos_t[bb])
        outs.append(sc_gather(h[bb], idx_b.reshape(N * KNN), base[bb]))
    out96 = jnp.stack(outs, axis=0)
    return jnp.concatenate([out96, x], axis=-1)


# speedup vs baseline: 2.6096x; 2.0450x over previous
import functools

import jax
import jax.numpy as jnp
from jax import lax
from jax.experimental import pallas as pl
from jax.experimental.pallas import tpu as pltpu
from jax.experimental.pallas import tpu_sc as plsc

B, N, D, KNN, GR = 4, 4096, 64, 16, 32
HD = 3 * GR
TILE = 256
NC, NS = 2, 16
NW = NC * NS
PPW = N // NW
CP = 8
NCHUNK = PPW // CP
SLOTS = 256
NP = N // SLOTS


def _hbase_body(x_ref, wf_ref, bf_ref, wm_ref, bm_ref, wl_ref, bl_ref,
                h_ref, base_ref):
    xb = x_ref[0]
    wf = wf_ref[...]
    wa, wb, wc = wf[0:D, :], wf[D:2 * D, :], wf[2 * D:3 * D, :]
    wm1, wm2 = wm_ref[0:GR, :], wm_ref[GR:GR + D, :]
    wl1, wl2, wl3 = wl_ref[0:GR, :], wl_ref[GR:2 * GR, :], wl_ref[2 * GR:, :]
    f32 = jnp.float32
    dot = functools.partial(jnp.dot, preferred_element_type=f32,
                            precision=lax.Precision.HIGHEST)
    u1 = wb + wc
    u2 = dot(u1, wm1)
    u3 = dot(u2, wl1) + dot(u1, wl2)
    a = dot(xb, wa - wc) + bf_ref[...]
    b = dot(a, wm1) + dot(xb, wm2) + bm_ref[...]
    c = dot(b, wl1) + dot(a, wl2) + dot(xb, wl3) + bl_ref[...]
    h_ref[0] = jnp.concatenate(
        [dot(xb, u3), dot(xb, u2), dot(xb, u1),
         jnp.zeros((N, 128 - HD), f32)], axis=1)
    base_ref[0] = jnp.concatenate([c, b, a], axis=1)


def _knn_body(post_ref, idx_ref):
    i = pl.program_id(0)
    rows = pl.ds(i * TILE, TILE)
    pos_t = post_ref[...]
    f32 = jnp.float32
    inf = jnp.float32(jnp.inf)

    d2_all = jnp.sum(pos_t * pos_t, axis=0)[None, :]
    pos_rows = post_ref[:, rows]
    d2_rows = jnp.sum(pos_rows * pos_rows, axis=0)[:, None]
    posb = pos_t.astype(jnp.bfloat16)
    pos_rows_b = pos_rows.astype(jnp.bfloat16)
    iota_s = lax.broadcasted_iota(jnp.int32, (TILE, SLOTS), 1)

    v1 = v2 = v3 = None
    for p in range(NP):
        csl = slice(p * SLOTS, (p + 1) * SLOTS)
        cross = lax.dot_general(pos_rows_b, posb[:, csl],
                                (((0,), (0,)), ((), ())),
                                preferred_element_type=f32)
        dpan = d2_rows + d2_all[:, csl] - 2.0 * cross
        ipan = iota_s + p * SLOTS
        if v1 is None:
            v1, i1 = dpan, ipan
            v2 = jnp.full((TILE, SLOTS), inf, f32)
            i2 = jnp.full((TILE, SLOTS), N, jnp.int32)
            v3, i3 = v2, i2
            continue
        lt1 = dpan < v1
        dv = jnp.where(lt1, v1, dpan)
        di = jnp.where(lt1, i1, ipan)
        v1 = jnp.where(lt1, dpan, v1)
        i1 = jnp.where(lt1, ipan, i1)
        lt2 = dv < v2
        dv2 = jnp.where(lt2, v2, dv)
        di2 = jnp.where(lt2, i2, di)
        v2 = jnp.where(lt2, dv, v2)
        i2 = jnp.where(lt2, di, i2)
        lt3 = dv2 < v3
        v3 = jnp.where(lt3, dv2, v3)
        i3 = jnp.where(lt3, di2, i3)

    V = jnp.concatenate([v1, v2, v3], axis=1)
    I = jnp.concatenate([i1, i2, i3], axis=1)

    rowmin = jnp.min(V, axis=1, keepdims=True)
    V = jnp.where(V == rowmin, inf, V)

    cols = []
    for _ in range(KNN):
        rowmin = jnp.min(V, axis=1, keepdims=True)
        eq = V == rowmin
        cand = jnp.where(eq, I, N)
        cols.append(jnp.min(cand, axis=1, keepdims=True))
        V = jnp.where(eq, inf, V)
    idx_ref[...] = jnp.concatenate(cols, axis=1)


def _sc_gather_body(h_hbm, idx_hbm, base_hbm, out_hbm, idx_v, rows_v, acc_v,
                    sem):
    wid = lax.axis_index("s") * NC + lax.axis_index("c")

    def chunk(ci, carry):
        point_base = wid * PPW + ci * CP
        pltpu.sync_copy(idx_hbm.at[pl.ds(point_base * KNN, CP * KNN)], idx_v)
        pltpu.async_copy(h_hbm.at[idx_v], rows_v, sem).wait()
        pltpu.sync_copy(base_hbm.at[pl.ds(point_base, CP)], acc_v)
        for p in range(CP):
            for c in range(HD // 16):
                sl = pl.ds(c * 16, 16)
                m = rows_v[p * KNN, sl]
                for n in range(1, KNN):
                    m = jnp.maximum(m, rows_v[p * KNN + n, sl])
                acc_v[p, sl] = acc_v[p, sl] + m
        pltpu.sync_copy(acc_v, out_hbm.at[pl.ds(point_base, CP)])
        return carry

    lax.fori_loop(0, NCHUNK, chunk, 0)


def kernel(x, pos, W_first, b_first, W_mid, b_mid, W_last, b_last):
    f32 = jnp.float32
    b_first2 = b_first.reshape(1, GR)
    b_mid2 = b_mid.reshape(1, GR)
    b_last2 = b_last.reshape(1, GR)

    h, base = pl.pallas_call(
        _hbase_body,
        grid=(B,),
        in_specs=[
            pl.BlockSpec((1, N, D), lambda b: (b, 0, 0)),
            pl.BlockSpec((3 * D, GR), lambda b: (0, 0)),
            pl.BlockSpec((1, GR), lambda b: (0, 0)),
            pl.BlockSpec((D + GR, GR), lambda b: (0, 0)),
            pl.BlockSpec((1, GR), lambda b: (0, 0)),
            pl.BlockSpec((D + 2 * GR, GR), lambda b: (0, 0)),
            pl.BlockSpec((1, GR), lambda b: (0, 0)),
        ],
        out_specs=[
            pl.BlockSpec((1, N, 128), lambda b: (b, 0, 0)),
            pl.BlockSpec((1, N, HD), lambda b: (b, 0, 0)),
        ],
        out_shape=[
            jax.ShapeDtypeStruct((B, N, 128), f32),
            jax.ShapeDtypeStruct((B, N, HD), f32),
        ],
    )(x, W_first, b_first2, W_mid, b_mid2, W_last, b_last2)

    pos_t = jnp.concatenate(
        [pos.transpose(0, 2, 1), jnp.zeros((B, 5, N), f32)], axis=1)

    knn_call = pl.pallas_call(
        _knn_body,
        grid=(N // TILE,),
        in_specs=[pl.BlockSpec((8, N), lambda i: (0, 0))],
        out_specs=pl.BlockSpec((TILE, KNN), lambda i: (i, 0)),
        out_shape=jax.ShapeDtypeStruct((N, KNN), jnp.int32),
        compiler_params=pltpu.CompilerParams(
            dimension_semantics=("arbitrary",)),
    )

    sc_gather = functools.partial(
        pl.kernel,
        out_type=jax.ShapeDtypeStruct((N, HD), f32),
        mesh=plsc.VectorSubcoreMesh(core_axis_name="c", subcore_axis_name="s"),
        scratch_types=[
            pltpu.VMEM((CP * KNN,), jnp.int32),
            pltpu.VMEM((CP * KNN, 128), f32),
            pltpu.VMEM((CP, HD), f32),
            pltpu.SemaphoreType.DMA,
        ],
    )(_sc_gather_body)

    outs = []
    for bb in range(B):
        idx_b = knn_call(pos_t[bb])
        outs.append(sc_gather(h[bb], idx_b.reshape(N * KNN), base[bb]))
    out96 = jnp.stack(outs, axis=0)
    return jnp.concatenate([out96, x], axis=-1)


# speedup vs baseline: 2.6184x; 1.0034x over previous
import functools

import jax
import jax.numpy as jnp
from jax import lax
from jax.experimental import pallas as pl
from jax.experimental.pallas import tpu as pltpu
from jax.experimental.pallas import tpu_sc as plsc

B, N, D, KNN, GR = 4, 4096, 64, 16, 32
HD = 3 * GR
TILE = 512
NC, NS = 2, 16
NW = NC * NS
PPW = N // NW
CP = 8
NCHUNK = PPW // CP
SLOTS = 256
NP = N // SLOTS


def _hbase_body(x_ref, wf_ref, bf_ref, wm_ref, bm_ref, wl_ref, bl_ref,
                h_ref, base_ref):
    xb = x_ref[0]
    wf = wf_ref[...]
    wa, wb, wc = wf[0:D, :], wf[D:2 * D, :], wf[2 * D:3 * D, :]
    wm1, wm2 = wm_ref[0:GR, :], wm_ref[GR:GR + D, :]
    wl1, wl2, wl3 = wl_ref[0:GR, :], wl_ref[GR:2 * GR, :], wl_ref[2 * GR:, :]
    f32 = jnp.float32
    dot = functools.partial(jnp.dot, preferred_element_type=f32,
                            precision=lax.Precision.HIGHEST)
    u1 = wb + wc
    u2 = dot(u1, wm1)
    u3 = dot(u2, wl1) + dot(u1, wl2)
    a = dot(xb, wa - wc) + bf_ref[...]
    b = dot(a, wm1) + dot(xb, wm2) + bm_ref[...]
    c = dot(b, wl1) + dot(a, wl2) + dot(xb, wl3) + bl_ref[...]
    h_ref[0] = jnp.concatenate(
        [dot(xb, u3), dot(xb, u2), dot(xb, u1),
         jnp.zeros((N, 128 - HD), f32)], axis=1)
    base_ref[0] = jnp.concatenate([c, b, a], axis=1)


def _knn_body(post_ref, idx_ref):
    i = pl.program_id(0)
    rows = pl.ds(i * TILE, TILE)
    pos_t = post_ref[...]
    f32 = jnp.float32
    inf = jnp.float32(jnp.inf)

    d2_all = jnp.sum(pos_t * pos_t, axis=0)[None, :]
    pos_rows = post_ref[:, rows]
    d2_rows = jnp.sum(pos_rows * pos_rows, axis=0)[:, None]
    posb = pos_t.astype(jnp.bfloat16)
    pos_rows_b = pos_rows.astype(jnp.bfloat16)
    iota_s = lax.broadcasted_iota(jnp.int32, (TILE, SLOTS), 1)

    v1 = v2 = v3 = None
    for p in range(NP):
        csl = slice(p * SLOTS, (p + 1) * SLOTS)
        cross = lax.dot_general(pos_rows_b, posb[:, csl],
                                (((0,), (0,)), ((), ())),
                                preferred_element_type=f32)
        dpan = d2_rows + d2_all[:, csl] - 2.0 * cross
        ipan = iota_s + p * SLOTS
        if v1 is None:
            v1, i1 = dpan, ipan
            v2 = jnp.full((TILE, SLOTS), inf, f32)
            i2 = jnp.full((TILE, SLOTS), N, jnp.int32)
            v3, i3 = v2, i2
            continue
        lt1 = dpan < v1
        dv = jnp.where(lt1, v1, dpan)
        di = jnp.where(lt1, i1, ipan)
        v1 = jnp.where(lt1, dpan, v1)
        i1 = jnp.where(lt1, ipan, i1)
        lt2 = dv < v2
        dv2 = jnp.where(lt2, v2, dv)
        di2 = jnp.where(lt2, i2, di)
        v2 = jnp.where(lt2, dv, v2)
        i2 = jnp.where(lt2, di, i2)
        lt3 = dv2 < v3
        v3 = jnp.where(lt3, dv2, v3)
        i3 = jnp.where(lt3, di2, i3)

    V = jnp.concatenate([v1, v2, v3], axis=1)
    I = jnp.concatenate([i1, i2, i3], axis=1)

    rowmin = jnp.min(V, axis=1, keepdims=True)
    V = jnp.where(V == rowmin, inf, V)

    cols = []
    for _ in range(KNN):
        rowmin = jnp.min(V, axis=1, keepdims=True)
        eq = V == rowmin
        cand = jnp.where(eq, I, N)
        cols.append(jnp.min(cand, axis=1, keepdims=True))
        V = jnp.where(eq, inf, V)
    idx_ref[...] = jnp.concatenate(cols, axis=1)


def _sc_gather_body(h_hbm, idx_hbm, base_hbm, out_hbm, idx_v, rows_v, acc_v,
                    sem):
    wid = lax.axis_index("s") * NC + lax.axis_index("c")

    def chunk(ci, carry):
        point_base = wid * PPW + ci * CP
        pltpu.sync_copy(idx_hbm.at[pl.ds(point_base * KNN, CP * KNN)], idx_v)
        pltpu.async_copy(h_hbm.at[idx_v], rows_v, sem).wait()
        pltpu.sync_copy(base_hbm.at[pl.ds(point_base, CP)], acc_v)
        for p in range(CP):
            for c in range(HD // 16):
                sl = pl.ds(c * 16, 16)
                m = rows_v[p * KNN, sl]
                for n in range(1, KNN):
                    m = jnp.maximum(m, rows_v[p * KNN + n, sl])
                acc_v[p, sl] = acc_v[p, sl] + m
        pltpu.sync_copy(acc_v, out_hbm.at[pl.ds(point_base, CP)])
        return carry

    lax.fori_loop(0, NCHUNK, chunk, 0)


def kernel(x, pos, W_first, b_first, W_mid, b_mid, W_last, b_last):
    f32 = jnp.float32
    b_first2 = b_first.reshape(1, GR)
    b_mid2 = b_mid.reshape(1, GR)
    b_last2 = b_last.reshape(1, GR)

    h, base = pl.pallas_call(
        _hbase_body,
        grid=(B,),
        in_specs=[
            pl.BlockSpec((1, N, D), lambda b: (b, 0, 0)),
            pl.BlockSpec((3 * D, GR), lambda b: (0, 0)),
            pl.BlockSpec((1, GR), lambda b: (0, 0)),
            pl.BlockSpec((D + GR, GR), lambda b: (0, 0)),
            pl.BlockSpec((1, GR), lambda b: (0, 0)),
            pl.BlockSpec((D + 2 * GR, GR), lambda b: (0, 0)),
            pl.BlockSpec((1, GR), lambda b: (0, 0)),
        ],
        out_specs=[
            pl.BlockSpec((1, N, 128), lambda b: (b, 0, 0)),
            pl.BlockSpec((1, N, HD), lambda b: (b, 0, 0)),
        ],
        out_shape=[
            jax.ShapeDtypeStruct((B, N, 128), f32),
            jax.ShapeDtypeStruct((B, N, HD), f32),
        ],
    )(x, W_first, b_first2, W_mid, b_mid2, W_last, b_last2)

    pos_t = jnp.concatenate(
        [pos.transpose(0, 2, 1), jnp.zeros((B, 5, N), f32)], axis=1)

    knn_call = pl.pallas_call(
        _knn_body,
        grid=(N // TILE,),
        in_specs=[pl.BlockSpec((8, N), lambda i: (0, 0))],
        out_specs=pl.BlockSpec((TILE, KNN), lambda i: (i, 0)),
        out_shape=jax.ShapeDtypeStruct((N, KNN), jnp.int32),
        compiler_params=pltpu.CompilerParams(
            dimension_semantics=("arbitrary",)),
    )

    sc_gather = functools.partial(
        pl.kernel,
        out_type=jax.ShapeDtypeStruct((N, HD), f32),
        mesh=plsc.VectorSubcoreMesh(core_axis_name="c", subcore_axis_name="s"),
        scratch_types=[
            pltpu.VMEM((CP * KNN,), jnp.int32),
            pltpu.VMEM((CP * KNN, 128), f32),
            pltpu.VMEM((CP, HD), f32),
            pltpu.SemaphoreType.DMA,
        ],
    )(_sc_gather_body)

    outs = []
    for bb in range(B):
        idx_b = knn_call(pos_t[bb])
        outs.append(sc_gather(h[bb], idx_b.reshape(N * KNN), base[bb]))
    out96 = jnp.stack(outs, axis=0)
    return jnp.concatenate([out96, x], axis=-1)


# speedup vs baseline: 2.7529x; 1.0514x over previous
import functools

import jax
import jax.numpy as jnp
from jax import lax
from jax.experimental import pallas as pl
from jax.experimental.pallas import tpu as pltpu
from jax.experimental.pallas import tpu_sc as plsc

B, N, D, KNN, GR = 4, 4096, 64, 16, 32
HD = 3 * GR
TILE = 512
NC, NS = 2, 16
NW = NC * NS
PPW = N // NW
CP = 8
NCHUNK = PPW // CP
SLOTS = 256
NP = N // SLOTS
NBUF = 4


def _hbase_body(x_ref, wf_ref, bf_ref, wm_ref, bm_ref, wl_ref, bl_ref,
                h_ref, base_ref):
    xb = x_ref[0]
    wf = wf_ref[...]
    wa, wb, wc = wf[0:D, :], wf[D:2 * D, :], wf[2 * D:3 * D, :]
    wm1, wm2 = wm_ref[0:GR, :], wm_ref[GR:GR + D, :]
    wl1, wl2, wl3 = wl_ref[0:GR, :], wl_ref[GR:2 * GR, :], wl_ref[2 * GR:, :]
    f32 = jnp.float32
    dot = functools.partial(jnp.dot, preferred_element_type=f32,
                            precision=lax.Precision.HIGHEST)
    u1 = wb + wc
    u2 = dot(u1, wm1)
    u3 = dot(u2, wl1) + dot(u1, wl2)
    a = dot(xb, wa - wc) + bf_ref[...]
    b = dot(a, wm1) + dot(xb, wm2) + bm_ref[...]
    c = dot(b, wl1) + dot(a, wl2) + dot(xb, wl3) + bl_ref[...]
    h_ref[0] = jnp.concatenate(
        [dot(xb, u3), dot(xb, u2), dot(xb, u1),
         jnp.zeros((N, 128 - HD), f32)], axis=1)
    base_ref[0] = jnp.concatenate([c, b, a], axis=1)


def _knn_body(post_ref, idx_ref):
    i = pl.program_id(0)
    rows = pl.ds(i * TILE, TILE)
    pos_t = post_ref[...]
    f32 = jnp.float32
    inf = jnp.float32(jnp.inf)

    d2_all = jnp.sum(pos_t * pos_t, axis=0)[None, :]
    pos_rows = post_ref[:, rows]
    d2_rows = jnp.sum(pos_rows * pos_rows, axis=0)[:, None]
    posb = pos_t.astype(jnp.bfloat16)
    pos_rows_b = pos_rows.astype(jnp.bfloat16)
    iota_s = lax.broadcasted_iota(jnp.int32, (TILE, SLOTS), 1)

    v1 = v2 = v3 = None
    for p in range(NP):
        csl = slice(p * SLOTS, (p + 1) * SLOTS)
        cross = lax.dot_general(pos_rows_b, posb[:, csl],
                                (((0,), (0,)), ((), ())),
                                preferred_element_type=f32)
        dpan = d2_rows + d2_all[:, csl] - 2.0 * cross
        ipan = iota_s + p * SLOTS
        if v1 is None:
            v1, i1 = dpan, ipan
            v2 = jnp.full((TILE, SLOTS), inf, f32)
            i2 = jnp.full((TILE, SLOTS), N, jnp.int32)
            v3, i3 = v2, i2
            continue
        lt1 = dpan < v1
        dv = jnp.where(lt1, v1, dpan)
        di = jnp.where(lt1, i1, ipan)
        v1 = jnp.where(lt1, dpan, v1)
        i1 = jnp.where(lt1, ipan, i1)
        lt2 = dv < v2
        dv2 = jnp.where(lt2, v2, dv)
        di2 = jnp.where(lt2, i2, di)
        v2 = jnp.where(lt2, dv, v2)
        i2 = jnp.where(lt2, di, i2)
        lt3 = dv2 < v3
        v3 = jnp.where(lt3, dv2, v3)
        i3 = jnp.where(lt3, di2, i3)

    V = jnp.concatenate([v1, v2, v3], axis=1)
    I = jnp.concatenate([i1, i2, i3], axis=1)

    rowmin = jnp.min(V, axis=1, keepdims=True)
    V = jnp.where(V == rowmin, inf, V)

    cols = []
    for _ in range(KNN):
        rowmin = jnp.min(V, axis=1, keepdims=True)
        eq = V == rowmin
        cand = jnp.where(eq, I, N)
        cols.append(jnp.min(cand, axis=1, keepdims=True))
        V = jnp.where(eq, inf, V)
    idx_ref[...] = jnp.concatenate(cols, axis=1)


def _sc_gather_body(h_hbm, idx_hbm, base_hbm, out_hbm,
                    idx_v0, idx_v1, idx_v2, idx_v3,
                    rows_v0, rows_v1, rows_v2, rows_v3,
                    acc_v, sem0, sem1, sem2, sem3):
    idx_bufs = (idx_v0, idx_v1, idx_v2, idx_v3)
    rows_bufs = (rows_v0, rows_v1, rows_v2, rows_v3)
    sems = (sem0, sem1, sem2, sem3)
    wid = lax.axis_index("s") * NC + lax.axis_index("c")

    def fire(c, b):
        point_base = wid * PPW + c * CP
        pltpu.sync_copy(idx_hbm.at[pl.ds(point_base * KNN, CP * KNN)],
                        idx_bufs[b])
        pltpu.async_copy(h_hbm.at[idx_bufs[b]], rows_bufs[b], sems[b])

    for b in range(NBUF):
        fire(b, b)

    def outer(gi, carry):
        for b in range(NBUF):
            c = gi * NBUF + b
            pltpu.make_async_copy(h_hbm.at[idx_bufs[b]], rows_bufs[b],
                                  sems[b]).wait()
            point_base = wid * PPW + c * CP
            pltpu.sync_copy(base_hbm.at[pl.ds(point_base, CP)], acc_v)
            rows_v = rows_bufs[b]

            def point(p, carry2):
                for ch in range(HD // 16):
                    sl = pl.ds(ch * 16, 16)
                    m = rows_v[p * KNN, sl]
                    for n in range(1, KNN):
                        m = jnp.maximum(m, rows_v[p * KNN + n, sl])
                    acc_v[p, sl] = acc_v[p, sl] + m
                return carry2

            lax.fori_loop(0, CP, point, 0)
            pltpu.sync_copy(acc_v, out_hbm.at[pl.ds(point_base, CP)])
            nc = c + NBUF

            @pl.when(nc < NCHUNK)
            def _():
                fire(nc, b)
        return carry

    lax.fori_loop(0, NCHUNK // NBUF, outer, 0)


def kernel(x, pos, W_first, b_first, W_mid, b_mid, W_last, b_last):
    f32 = jnp.float32
    b_first2 = b_first.reshape(1, GR)
    b_mid2 = b_mid.reshape(1, GR)
    b_last2 = b_last.reshape(1, GR)

    h, base = pl.pallas_call(
        _hbase_body,
        grid=(B,),
        in_specs=[
            pl.BlockSpec((1, N, D), lambda b: (b, 0, 0)),
            pl.BlockSpec((3 * D, GR), lambda b: (0, 0)),
            pl.BlockSpec((1, GR), lambda b: (0, 0)),
            pl.BlockSpec((D + GR, GR), lambda b: (0, 0)),
            pl.BlockSpec((1, GR), lambda b: (0, 0)),
            pl.BlockSpec((D + 2 * GR, GR), lambda b: (0, 0)),
            pl.BlockSpec((1, GR), lambda b: (0, 0)),
        ],
        out_specs=[
            pl.BlockSpec((1, N, 128), lambda b: (b, 0, 0)),
            pl.BlockSpec((1, N, HD), lambda b: (b, 0, 0)),
        ],
        out_shape=[
            jax.ShapeDtypeStruct((B, N, 128), f32),
            jax.ShapeDtypeStruct((B, N, HD), f32),
        ],
    )(x, W_first, b_first2, W_mid, b_mid2, W_last, b_last2)

    pos_t = jnp.concatenate(
        [pos.transpose(0, 2, 1), jnp.zeros((B, 5, N), f32)], axis=1)

    knn_call = pl.pallas_call(
        _knn_body,
        grid=(N // TILE,),
        in_specs=[pl.BlockSpec((8, N), lambda i: (0, 0))],
        out_specs=pl.BlockSpec((TILE, KNN), lambda i: (i, 0)),
        out_shape=jax.ShapeDtypeStruct((N, KNN), jnp.int32),
        compiler_params=pltpu.CompilerParams(
            dimension_semantics=("arbitrary",)),
    )

    sc_gather = functools.partial(
        pl.kernel,
        out_type=jax.ShapeDtypeStruct((N, HD), f32),
        mesh=plsc.VectorSubcoreMesh(core_axis_name="c", subcore_axis_name="s"),
        scratch_types=(
            [pltpu.VMEM((CP * KNN,), jnp.int32) for _ in range(NBUF)]
            + [pltpu.VMEM((CP * KNN, 128), f32) for _ in range(NBUF)]
            + [pltpu.VMEM((CP, HD), f32)]
            + [pltpu.SemaphoreType.DMA for _ in range(NBUF)]
        ),
    )(_sc_gather_body)

    outs = []
    for bb in range(B):
        idx_b = knn_call(pos_t[bb])
        outs.append(sc_gather(h[bb], idx_b.reshape(N * KNN), base[bb]))
    out96 = jnp.stack(outs, axis=0)
    return jnp.concatenate([out96, x], axis=-1)


# speedup vs baseline: 2.9807x; 1.0827x over previous
import functools

import jax
import jax.numpy as jnp
from jax import lax
from jax.experimental import pallas as pl
from jax.experimental.pallas import tpu as pltpu
from jax.experimental.pallas import tpu_sc as plsc

B, N, D, KNN, GR = 4, 4096, 64, 16, 32
HD = 3 * GR
TILE = 512
NC, NS = 2, 16
NW = NC * NS
PPW = N // NW
CP = 8
NCHUNK = PPW // CP
SLOTS = 256
NP = N // SLOTS
NBUF = 4


def _hbase_body(x_ref, wf_ref, bf_ref, wm_ref, bm_ref, wl_ref, bl_ref,
                h_ref, base_ref):
    xb = x_ref[0]
    wf = wf_ref[...]
    wa, wb, wc = wf[0:D, :], wf[D:2 * D, :], wf[2 * D:3 * D, :]
    wm1, wm2 = wm_ref[0:GR, :], wm_ref[GR:GR + D, :]
    wl1, wl2, wl3 = wl_ref[0:GR, :], wl_ref[GR:2 * GR, :], wl_ref[2 * GR:, :]
    f32 = jnp.float32
    dot = functools.partial(jnp.dot, preferred_element_type=f32,
                            precision=lax.Precision.HIGHEST)
    u1 = wb + wc
    u2 = dot(u1, wm1)
    u3 = dot(u2, wl1) + dot(u1, wl2)
    ma, ka = wa - wc, bf_ref[...]
    mb, kb = dot(ma, wm1) + wm2, dot(ka, wm1) + bm_ref[...]
    mc = dot(mb, wl1) + dot(ma, wl2) + wl3
    kc = dot(kb, wl1) + dot(ka, wl2) + bl_ref[...]
    u128 = jnp.concatenate([u3, u2, u1, jnp.zeros((D, 128 - HD), f32)],
                           axis=1)
    m96 = jnp.concatenate([mc, mb, ma], axis=1)
    k96 = jnp.concatenate([kc, kb, ka], axis=1)
    h_ref[0] = dot(xb, u128)
    base_ref[0] = dot(xb, m96) + k96


def _knn_body(post_ref, idx_ref):
    i = pl.program_id(0)
    rows = pl.ds(i * TILE, TILE)
    pos_t = post_ref[...]
    f32 = jnp.float32
    inf = jnp.float32(jnp.inf)

    d2_all = jnp.sum(pos_t * pos_t, axis=0)[None, :]
    pos_rows = post_ref[:, rows]
    d2_rows = jnp.sum(pos_rows * pos_rows, axis=0)[:, None]
    posb = pos_t.astype(jnp.bfloat16)
    pos_rows_b = pos_rows.astype(jnp.bfloat16)
    iota_s = lax.broadcasted_iota(jnp.int32, (TILE, SLOTS), 1)

    v1 = v2 = v3 = None
    for p in range(NP):
        csl = slice(p * SLOTS, (p + 1) * SLOTS)
        cross = lax.dot_general(pos_rows_b, posb[:, csl],
                                (((0,), (0,)), ((), ())),
                                preferred_element_type=f32)
        dpan = d2_rows + d2_all[:, csl] - 2.0 * cross
        ipan = iota_s + p * SLOTS
        if v1 is None:
            v1, i1 = dpan, ipan
            v2 = jnp.full((TILE, SLOTS), inf, f32)
            i2 = jnp.full((TILE, SLOTS), N, jnp.int32)
            v3, i3 = v2, i2
            continue
        lt1 = dpan < v1
        dv = jnp.where(lt1, v1, dpan)
        di = jnp.where(lt1, i1, ipan)
        v1 = jnp.where(lt1, dpan, v1)
        i1 = jnp.where(lt1, ipan, i1)
        lt2 = dv < v2
        dv2 = jnp.where(lt2, v2, dv)
        di2 = jnp.where(lt2, i2, di)
        v2 = jnp.where(lt2, dv, v2)
        i2 = jnp.where(lt2, di, i2)
        lt3 = dv2 < v3
        v3 = jnp.where(lt3, dv2, v3)
        i3 = jnp.where(lt3, di2, i3)

    V = jnp.concatenate([v1, v2, v3], axis=1)
    I = jnp.concatenate([i1, i2, i3], axis=1)

    rowmin = jnp.min(V, axis=1, keepdims=True)
    V = jnp.where(V == rowmin, inf, V)

    cols = []
    for _ in range(KNN):
        rowmin = jnp.min(V, axis=1, keepdims=True)
        eq = V == rowmin
        cand = jnp.where(eq, I, N)
        cols.append(jnp.min(cand, axis=1, keepdims=True))
        V = jnp.where(eq, inf, V)
    idx_ref[...] = jnp.concatenate(cols, axis=1)


def _sc_gather_body(h_hbm, idx_hbm, base_hbm, x_hbm, out_hbm,
                    idx_v0, idx_v1, idx_v2, idx_v3,
                    rows_v0, rows_v1, rows_v2, rows_v3,
                    acc_v, base_v, x_v, sem0, sem1, sem2, sem3):
    idx_bufs = (idx_v0, idx_v1, idx_v2, idx_v3)
    rows_bufs = (rows_v0, rows_v1, rows_v2, rows_v3)
    sems = (sem0, sem1, sem2, sem3)
    wid = lax.axis_index("s") * NC + lax.axis_index("c")

    def fire(c, b):
        point_base = wid * PPW + c * CP
        pltpu.sync_copy(idx_hbm.at[pl.ds(point_base * KNN, CP * KNN)],
                        idx_bufs[b])
        pltpu.async_copy(h_hbm.at[idx_bufs[b]], rows_bufs[b], sems[b])

    for b in range(NBUF):
        fire(b, b)

    def outer(gi, carry):
        for b in range(NBUF):
            c = gi * NBUF + b
            pltpu.make_async_copy(h_hbm.at[idx_bufs[b]], rows_bufs[b],
                                  sems[b]).wait()
            point_base = wid * PPW + c * CP
            pltpu.sync_copy(base_hbm.at[pl.ds(point_base, CP)], base_v)
            pltpu.sync_copy(x_hbm.at[pl.ds(point_base, CP)], x_v)
            rows_v = rows_bufs[b]

            def point(p, carry2):
                for ch in range(HD // 16):
                    sl = pl.ds(ch * 16, 16)
                    m = rows_v[p * KNN, sl]
                    for n in range(1, KNN):
                        m = jnp.maximum(m, rows_v[p * KNN + n, sl])
                    acc_v[p, sl] = base_v[p, sl] + m
                for ch in range(D // 16):
                    sl = pl.ds(ch * 16, 16)
                    acc_v[p, pl.ds(HD + ch * 16, 16)] = x_v[p, sl]
                return carry2

            lax.fori_loop(0, CP, point, 0)
            pltpu.sync_copy(acc_v, out_hbm.at[pl.ds(point_base, CP)])
            nc = c + NBUF

            @pl.when(nc < NCHUNK)
            def _():
                fire(nc, b)
        return carry

    lax.fori_loop(0, NCHUNK // NBUF, outer, 0)


def kernel(x, pos, W_first, b_first, W_mid, b_mid, W_last, b_last):
    f32 = jnp.float32
    b_first2 = b_first.reshape(1, GR)
    b_mid2 = b_mid.reshape(1, GR)
    b_last2 = b_last.reshape(1, GR)

    h, base = pl.pallas_call(
        _hbase_body,
        grid=(B,),
        in_specs=[
            pl.BlockSpec((1, N, D), lambda b: (b, 0, 0)),
            pl.BlockSpec((3 * D, GR), lambda b: (0, 0)),
            pl.BlockSpec((1, GR), lambda b: (0, 0)),
            pl.BlockSpec((D + GR, GR), lambda b: (0, 0)),
            pl.BlockSpec((1, GR), lambda b: (0, 0)),
            pl.BlockSpec((D + 2 * GR, GR), lambda b: (0, 0)),
            pl.BlockSpec((1, GR), lambda b: (0, 0)),
        ],
        out_specs=[
            pl.BlockSpec((1, N, 128), lambda b: (b, 0, 0)),
            pl.BlockSpec((1, N, HD), lambda b: (b, 0, 0)),
        ],
        out_shape=[
            jax.ShapeDtypeStruct((B, N, 128), f32),
            jax.ShapeDtypeStruct((B, N, HD), f32),
        ],
    )(x, W_first, b_first2, W_mid, b_mid2, W_last, b_last2)

    pos_t = jnp.concatenate(
        [pos.transpose(0, 2, 1), jnp.zeros((B, 5, N), f32)], axis=1)

    knn_call = pl.pallas_call(
        _knn_body,
        grid=(N // TILE,),
        in_specs=[pl.BlockSpec((8, N), lambda i: (0, 0))],
        out_specs=pl.BlockSpec((TILE, KNN), lambda i: (i, 0)),
        out_shape=jax.ShapeDtypeStruct((N, KNN), jnp.int32),
        compiler_params=pltpu.CompilerParams(
            dimension_semantics=("arbitrary",)),
    )

    sc_gather = functools.partial(
        pl.kernel,
        out_type=jax.ShapeDtypeStruct((N, HD + D), f32),
        mesh=plsc.VectorSubcoreMesh(core_axis_name="c", subcore_axis_name="s"),
        scratch_types=(
            [pltpu.VMEM((CP * KNN,), jnp.int32) for _ in range(NBUF)]
            + [pltpu.VMEM((CP * KNN, 128), f32) for _ in range(NBUF)]
            + [pltpu.VMEM((CP, HD + D), f32), pltpu.VMEM((CP, HD), f32),
               pltpu.VMEM((CP, D), f32)]
            + [pltpu.SemaphoreType.DMA for _ in range(NBUF)]
        ),
    )(_sc_gather_body)

    outs = []
    for bb in range(B):
        idx_b = knn_call(pos_t[bb])
        outs.append(sc_gather(h[bb], idx_b.reshape(N * KNN), base[bb], x[bb]))
    return jnp.stack(outs, axis=0)


# speedup vs baseline: 3.8846x; 1.3033x over previous
import functools

import jax
import jax.numpy as jnp
from jax import lax
from jax.experimental import pallas as pl
from jax.experimental.pallas import tpu as pltpu
from jax.experimental.pallas import tpu_sc as plsc

B, N, D, KNN, GR = 4, 4096, 64, 16, 32
HD = 3 * GR
TILE = 512
NC, NS = 2, 16
NW = NC * NS
PPW = N // NW
CP = 8
NCHUNK = PPW // CP
SLOTS = 256
NP = N // SLOTS
NBUF = 4


def _hbase_body(x_ref, wf_ref, bf_ref, wm_ref, bm_ref, wl_ref, bl_ref,
                h_ref, base_ref):
    xb = x_ref[0]
    wf = wf_ref[...]
    wa, wb, wc = wf[0:D, :], wf[D:2 * D, :], wf[2 * D:3 * D, :]
    wm1, wm2 = wm_ref[0:GR, :], wm_ref[GR:GR + D, :]
    wl1, wl2, wl3 = wl_ref[0:GR, :], wl_ref[GR:2 * GR, :], wl_ref[2 * GR:, :]
    f32 = jnp.float32
    dot = functools.partial(jnp.dot, preferred_element_type=f32,
                            precision=lax.Precision.HIGHEST)
    u1 = wb + wc
    u2 = dot(u1, wm1)
    u3 = dot(u2, wl1) + dot(u1, wl2)
    ma, ka = wa - wc, bf_ref[...]
    mb, kb = dot(ma, wm1) + wm2, dot(ka, wm1) + bm_ref[...]
    mc = dot(mb, wl1) + dot(ma, wl2) + wl3
    kc = dot(kb, wl1) + dot(ka, wl2) + bl_ref[...]
    u128 = jnp.concatenate([u3, u2, u1, jnp.zeros((D, 128 - HD), f32)],
                           axis=1)
    m96 = jnp.concatenate([mc, mb, ma], axis=1)
    k96 = jnp.concatenate([kc, kb, ka], axis=1)
    h_ref[0] = dot(xb, u128)
    base_ref[0] = dot(xb, m96) + k96


def _knn_body(post_ref, idx_ref):
    i = pl.program_id(0)
    rows = pl.ds(i * TILE, TILE)
    pos_t = post_ref[...]
    f32 = jnp.float32
    inf = jnp.float32(jnp.inf)

    d2_all = jnp.sum(pos_t * pos_t, axis=0)[None, :]
    pos_rows = post_ref[:, rows]
    d2_rows = jnp.sum(pos_rows * pos_rows, axis=0)[:, None]
    posb = pos_t.astype(jnp.bfloat16)
    pos_rows_b = pos_rows.astype(jnp.bfloat16)

    mask_i = jnp.int32(-16)
    v1 = v2 = v3 = None
    for p in range(NP):
        csl = slice(p * SLOTS, (p + 1) * SLOTS)
        cross = lax.dot_general(pos_rows_b, posb[:, csl],
                                (((0,), (0,)), ((), ())),
                                preferred_element_type=f32)
        dpan = d2_rows + d2_all[:, csl] - 2.0 * cross
        key = lax.bitcast_convert_type(
            (lax.bitcast_convert_type(dpan, jnp.int32) & mask_i)
            | jnp.int32(p), f32)
        if v1 is None:
            v1 = key
            v2 = jnp.full((TILE, SLOTS), inf, f32)
            v3 = v2
            continue
        d1 = jnp.maximum(v1, key)
        v1 = jnp.minimum(v1, key)
        d2 = jnp.maximum(v2, d1)
        v2 = jnp.minimum(v2, d1)
        v3 = jnp.minimum(v3, d2)

    V = jnp.concatenate([v1, v2, v3], axis=1)
    iota_f = lax.broadcasted_iota(
        jnp.int32, (TILE, SLOTS), 1).astype(f32)
    If = jnp.concatenate([iota_f, iota_f, iota_f], axis=1)

    rowmin = jnp.min(V, axis=1, keepdims=True)
    V = jnp.where(V == rowmin, inf, V)

    cols = []
    for _ in range(KNN):
        rowmin = jnp.min(V, axis=1, keepdims=True)
        eq = V == rowmin
        cand = jnp.where(eq, If, jnp.float32(N))
        slot = jnp.min(cand, axis=1, keepdims=True).astype(jnp.int32)
        panel = lax.bitcast_convert_type(rowmin, jnp.int32) & jnp.int32(0xF)
        cols.append(panel * SLOTS + slot)
        V = jnp.where(eq, inf, V)
    idx_ref[...] = jnp.concatenate(cols, axis=1)


def _sc_gather_body(h_hbm, idx_hbm, base_hbm, x_hbm, out_hbm,
                    idx_v0, idx_v1, idx_v2, idx_v3,
                    rows_v0, rows_v1, rows_v2, rows_v3,
                    acc_v, base_v, x_v, sem0, sem1, sem2, sem3):
    idx_bufs = (idx_v0, idx_v1, idx_v2, idx_v3)
    rows_bufs = (rows_v0, rows_v1, rows_v2, rows_v3)
    sems = (sem0, sem1, sem2, sem3)
    wid = lax.axis_index("s") * NC + lax.axis_index("c")

    def fire(c, b):
        point_base = wid * PPW + c * CP
        pltpu.sync_copy(idx_hbm.at[pl.ds(point_base * KNN, CP * KNN)],
                        idx_bufs[b])
        pltpu.async_copy(h_hbm.at[idx_bufs[b]], rows_bufs[b], sems[b])

    for b in range(NBUF):
        fire(b, b)

    def outer(gi, carry):
        for b in range(NBUF):
            c = gi * NBUF + b
            pltpu.make_async_copy(h_hbm.at[idx_bufs[b]], rows_bufs[b],
                                  sems[b]).wait()
            point_base = wid * PPW + c * CP
            pltpu.sync_copy(base_hbm.at[pl.ds(point_base, CP)], base_v)
            pltpu.sync_copy(x_hbm.at[pl.ds(point_base, CP)], x_v)
            rows_v = rows_bufs[b]

            def point(p, carry2):
                for ch in range(HD // 16):
                    sl = pl.ds(ch * 16, 16)
                    m = rows_v[p * KNN, sl]
                    for n in range(1, KNN):
                        m = jnp.maximum(m, rows_v[p * KNN + n, sl])
                    acc_v[p, sl] = base_v[p, sl] + m
                for ch in range(D // 16):
                    sl = pl.ds(ch * 16, 16)
                    acc_v[p, pl.ds(HD + ch * 16, 16)] = x_v[p, sl]
                return carry2

            lax.fori_loop(0, CP, point, 0)
            pltpu.sync_copy(acc_v, out_hbm.at[pl.ds(point_base, CP)])
            nc = c + NBUF

            @pl.when(nc < NCHUNK)
            def _():
                fire(nc, b)
        return carry

    lax.fori_loop(0, NCHUNK // NBUF, outer, 0)


def kernel(x, pos, W_first, b_first, W_mid, b_mid, W_last, b_last):
    f32 = jnp.float32
    b_first2 = b_first.reshape(1, GR)
    b_mid2 = b_mid.reshape(1, GR)
    b_last2 = b_last.reshape(1, GR)

    h, base = pl.pallas_call(
        _hbase_body,
        grid=(B,),
        in_specs=[
            pl.BlockSpec((1, N, D), lambda b: (b, 0, 0)),
            pl.BlockSpec((3 * D, GR), lambda b: (0, 0)),
            pl.BlockSpec((1, GR), lambda b: (0, 0)),
            pl.BlockSpec((D + GR, GR), lambda b: (0, 0)),
            pl.BlockSpec((1, GR), lambda b: (0, 0)),
            pl.BlockSpec((D + 2 * GR, GR), lambda b: (0, 0)),
            pl.BlockSpec((1, GR), lambda b: (0, 0)),
        ],
        out_specs=[
            pl.BlockSpec((1, N, 128), lambda b: (b, 0, 0)),
            pl.BlockSpec((1, N, HD), lambda b: (b, 0, 0)),
        ],
        out_shape=[
            jax.ShapeDtypeStruct((B, N, 128), f32),
            jax.ShapeDtypeStruct((B, N, HD), f32),
        ],
    )(x, W_first, b_first2, W_mid, b_mid2, W_last, b_last2)

    pos_t = jnp.concatenate(
        [pos.transpose(0, 2, 1), jnp.zeros((B, 5, N), f32)], axis=1)

    knn_call = pl.pallas_call(
        _knn_body,
        grid=(N // TILE,),
        in_specs=[pl.BlockSpec((8, N), lambda i: (0, 0))],
        out_specs=pl.BlockSpec((TILE, KNN), lambda i: (i, 0)),
        out_shape=jax.ShapeDtypeStruct((N, KNN), jnp.int32),
        compiler_params=pltpu.CompilerParams(
            dimension_semantics=("arbitrary",)),
    )

    sc_gather = functools.partial(
        pl.kernel,
        out_type=jax.ShapeDtypeStruct((N, HD + D), f32),
        mesh=plsc.VectorSubcoreMesh(core_axis_name="c", subcore_axis_name="s"),
        scratch_types=(
            [pltpu.VMEM((CP * KNN,), jnp.int32) for _ in range(NBUF)]
            + [pltpu.VMEM((CP * KNN, 128), f32) for _ in range(NBUF)]
            + [pltpu.VMEM((CP, HD + D), f32), pltpu.VMEM((CP, HD), f32),
               pltpu.VMEM((CP, D), f32)]
            + [pltpu.SemaphoreType.DMA for _ in range(NBUF)]
        ),
    )(_sc_gather_body)

    outs = []
    for bb in range(B):
        idx_b = knn_call(pos_t[bb])
        outs.append(sc_gather(h[bb], idx_b.reshape(N * KNN), base[bb], x[bb]))
    return jnp.stack(outs, axis=0)


# speedup vs baseline: 4.0640x; 1.0462x over previous
import functools

import jax
import jax.numpy as jnp
from jax import lax
from jax.experimental import pallas as pl
from jax.experimental.pallas import tpu as pltpu
from jax.experimental.pallas import tpu_sc as plsc

B, N, D, KNN, GR = 4, 4096, 64, 16, 32
HD = 3 * GR
TILE = 512
NC, NS = 2, 16
NW = NC * NS
PPW = N // NW
CP = 8
NCHUNK = PPW // CP
SLOTS = 128
NP = N // SLOTS
NBUF = 4


def _hbase_body(x_ref, wf_ref, bf_ref, wm_ref, bm_ref, wl_ref, bl_ref,
                h_ref, base_ref):
    xb = x_ref[0]
    wf = wf_ref[...]
    wa, wb, wc = wf[0:D, :], wf[D:2 * D, :], wf[2 * D:3 * D, :]
    wm1, wm2 = wm_ref[0:GR, :], wm_ref[GR:GR + D, :]
    wl1, wl2, wl3 = wl_ref[0:GR, :], wl_ref[GR:2 * GR, :], wl_ref[2 * GR:, :]
    f32 = jnp.float32
    dot = functools.partial(jnp.dot, preferred_element_type=f32,
                            precision=lax.Precision.HIGHEST)
    u1 = wb + wc
    u2 = dot(u1, wm1)
    u3 = dot(u2, wl1) + dot(u1, wl2)
    ma, ka = wa - wc, bf_ref[...]
    mb, kb = dot(ma, wm1) + wm2, dot(ka, wm1) + bm_ref[...]
    mc = dot(mb, wl1) + dot(ma, wl2) + wl3
    kc = dot(kb, wl1) + dot(ka, wl2) + bl_ref[...]
    u128 = jnp.concatenate([u3, u2, u1, jnp.zeros((D, 128 - HD), f32)],
                           axis=1)
    m96 = jnp.concatenate([mc, mb, ma], axis=1)
    k96 = jnp.concatenate([kc, kb, ka], axis=1)
    h_ref[0] = dot(xb, u128)
    base_ref[0] = dot(xb, m96) + k96


def _knn_body(post_ref, idx_ref):
    i = pl.program_id(0)
    rows = pl.ds(i * TILE, TILE)
    pos_t = post_ref[...]
    f32 = jnp.float32
    inf = jnp.float32(jnp.inf)

    d2_all = jnp.sum(pos_t * pos_t, axis=0)[None, :]
    pos_rows = post_ref[:, rows]
    d2_rows = jnp.sum(pos_rows * pos_rows, axis=0)[:, None]
    posb = pos_t.astype(jnp.bfloat16)
    pos_rows_b = pos_rows.astype(jnp.bfloat16)

    mask_i = jnp.int32(-32)
    v1 = v2 = v3 = v4 = None
    for p in range(NP):
        csl = slice(p * SLOTS, (p + 1) * SLOTS)
        cross = lax.dot_general(pos_rows_b, posb[:, csl],
                                (((0,), (0,)), ((), ())),
                                preferred_element_type=f32)
        dpan = d2_rows + d2_all[:, csl] - 2.0 * cross
        key = lax.bitcast_convert_type(
            (lax.bitcast_convert_type(dpan, jnp.int32) & mask_i)
            | jnp.int32(p), f32)
        if v1 is None:
            v1 = key
            v2 = jnp.full((TILE, SLOTS), inf, f32)
            v3 = v4 = v2
            continue
        d1 = jnp.maximum(v1, key)
        v1 = jnp.minimum(v1, key)
        d2 = jnp.maximum(v2, d1)
        v2 = jnp.minimum(v2, d1)
        d3 = jnp.maximum(v3, d2)
        v3 = jnp.minimum(v3, d2)
        v4 = jnp.minimum(v4, d3)

    V = jnp.concatenate([v1, v2, v3, v4], axis=1)
    iota_f = lax.broadcasted_iota(
        jnp.int32, (TILE, SLOTS), 1).astype(f32)
    If = jnp.concatenate([iota_f, iota_f, iota_f, iota_f], axis=1)

    rowmin = jnp.min(V, axis=1, keepdims=True)
    V = jnp.where(V == rowmin, inf, V)

    cols = []
    for _ in range(KNN):
        rowmin = jnp.min(V, axis=1, keepdims=True)
        eq = V == rowmin
        cand = jnp.where(eq, If, jnp.float32(N))
        slot = jnp.min(cand, axis=1, keepdims=True).astype(jnp.int32)
        panel = lax.bitcast_convert_type(rowmin, jnp.int32) & jnp.int32(0x1F)
        cols.append(panel * SLOTS + slot)
        V = jnp.where(eq, inf, V)
    idx_ref[...] = jnp.concatenate(cols, axis=1)


def _sc_gather_body(h_hbm, idx_hbm, base_hbm, x_hbm, out_hbm,
                    idx_v0, idx_v1, idx_v2, idx_v3,
                    rows_v0, rows_v1, rows_v2, rows_v3,
                    acc_v, base_v, x_v, sem0, sem1, sem2, sem3):
    idx_bufs = (idx_v0, idx_v1, idx_v2, idx_v3)
    rows_bufs = (rows_v0, rows_v1, rows_v2, rows_v3)
    sems = (sem0, sem1, sem2, sem3)
    wid = lax.axis_index("s") * NC + lax.axis_index("c")

    def fire(c, b):
        point_base = wid * PPW + c * CP
        pltpu.sync_copy(idx_hbm.at[pl.ds(point_base * KNN, CP * KNN)],
                        idx_bufs[b])
        pltpu.async_copy(h_hbm.at[idx_bufs[b]], rows_bufs[b], sems[b])

    for b in range(NBUF):
        fire(b, b)

    def outer(gi, carry):
        for b in range(NBUF):
            c = gi * NBUF + b
            pltpu.make_async_copy(h_hbm.at[idx_bufs[b]], rows_bufs[b],
                                  sems[b]).wait()
            point_base = wid * PPW + c * CP
            pltpu.sync_copy(base_hbm.at[pl.ds(point_base, CP)], base_v)
            pltpu.sync_copy(x_hbm.at[pl.ds(point_base, CP)], x_v)
            rows_v = rows_bufs[b]

            def point(p, carry2):
                for ch in range(HD // 16):
                    sl = pl.ds(ch * 16, 16)
                    m = rows_v[p * KNN, sl]
                    for n in range(1, KNN):
                        m = jnp.maximum(m, rows_v[p * KNN + n, sl])
                    acc_v[p, sl] = base_v[p, sl] + m
                for ch in range(D // 16):
                    sl = pl.ds(ch * 16, 16)
                    acc_v[p, pl.ds(HD + ch * 16, 16)] = x_v[p, sl]
                return carry2

            lax.fori_loop(0, CP, point, 0)
            pltpu.sync_copy(acc_v, out_hbm.at[pl.ds(point_base, CP)])
            nc = c + NBUF

            @pl.when(nc < NCHUNK)
            def _():
                fire(nc, b)
        return carry

    lax.fori_loop(0, NCHUNK // NBUF, outer, 0)


def kernel(x, pos, W_first, b_first, W_mid, b_mid, W_last, b_last):
    f32 = jnp.float32
    b_first2 = b_first.reshape(1, GR)
    b_mid2 = b_mid.reshape(1, GR)
    b_last2 = b_last.reshape(1, GR)

    h, base = pl.pallas_call(
        _hbase_body,
        grid=(B,),
        in_specs=[
            pl.BlockSpec((1, N, D), lambda b: (b, 0, 0)),
            pl.BlockSpec((3 * D, GR), lambda b: (0, 0)),
            pl.BlockSpec((1, GR), lambda b: (0, 0)),
            pl.BlockSpec((D + GR, GR), lambda b: (0, 0)),
            pl.BlockSpec((1, GR), lambda b: (0, 0)),
            pl.BlockSpec((D + 2 * GR, GR), lambda b: (0, 0)),
            pl.BlockSpec((1, GR), lambda b: (0, 0)),
        ],
        out_specs=[
            pl.BlockSpec((1, N, 128), lambda b: (b, 0, 0)),
            pl.BlockSpec((1, N, HD), lambda b: (b, 0, 0)),
        ],
        out_shape=[
            jax.ShapeDtypeStruct((B, N, 128), f32),
            jax.ShapeDtypeStruct((B, N, HD), f32),
        ],
    )(x, W_first, b_first2, W_mid, b_mid2, W_last, b_last2)

    pos_t = jnp.concatenate(
        [pos.transpose(0, 2, 1), jnp.zeros((B, 5, N), f32)], axis=1)

    knn_call = pl.pallas_call(
        _knn_body,
        grid=(N // TILE,),
        in_specs=[pl.BlockSpec((8, N), lambda i: (0, 0))],
        out_specs=pl.BlockSpec((TILE, KNN), lambda i: (i, 0)),
        out_shape=jax.ShapeDtypeStruct((N, KNN), jnp.int32),
        compiler_params=pltpu.CompilerParams(
            dimension_semantics=("arbitrary",)),
    )

    sc_gather = functools.partial(
        pl.kernel,
        out_type=jax.ShapeDtypeStruct((N, HD + D), f32),
        mesh=plsc.VectorSubcoreMesh(core_axis_name="c", subcore_axis_name="s"),
        scratch_types=(
            [pltpu.VMEM((CP * KNN,), jnp.int32) for _ in range(NBUF)]
            + [pltpu.VMEM((CP * KNN, 128), f32) for _ in range(NBUF)]
            + [pltpu.VMEM((CP, HD + D), f32), pltpu.VMEM((CP, HD), f32),
               pltpu.VMEM((CP, D), f32)]
            + [pltpu.SemaphoreType.DMA for _ in range(NBUF)]
        ),
    )(_sc_gather_body)

    outs = []
    for bb in range(B):
        idx_b = knn_call(pos_t[bb])
        outs.append(sc_gather(h[bb], idx_b.reshape(N * KNN), base[bb], x[bb]))
    return jnp.stack(outs, axis=0)
